# baseline (speedup 1.0000x reference)
import functools

import jax
import jax.numpy as jnp
from jax import lax
from jax.experimental import pallas as pl
from jax.experimental.pallas import tpu as pltpu
from jax.experimental.pallas import tpu_sc as plsc

K = 8
D = 128
T = 262144

NC = 2
NS = 16
NW = NC * NS
TOK_PER_W = T // NW
CHUNK = 128
NCHUNK = TOK_PER_W // CHUNK


def _proj_body(table_ref, w_ref, b_ref, out_ref):
    out_ref[...] = (
        jnp.dot(table_ref[...], w_ref[...], preferred_element_type=jnp.float32)
        + b_ref[...]
    )


def _project_table(table, W, b):
    return pl.pallas_call(
        _proj_body,
        out_shape=jax.ShapeDtypeStruct((K, D), jnp.float32),
    )(table, W, b.reshape(1, D))


_sc_mesh = plsc.VectorSubcoreMesh(
    core_axis_name="c", subcore_axis_name="s", num_cores=NC, num_subcores=NS
)

NBUF = 4
LAG = 2


@functools.partial(
    pl.kernel,
    out_type=jax.ShapeDtypeStruct((T, D), jnp.float32),
    mesh=_sc_mesh,
    scratch_types=[
        pltpu.VMEM((NCHUNK, CHUNK), jnp.int32),
        [pltpu.VMEM((CHUNK, D), jnp.float32)] * NBUF,
        [pltpu.SemaphoreType.DMA] * NBUF,
        [pltpu.SemaphoreType.DMA] * NBUF,
    ],
)
def _sc_gather(p_hbm, tok_hbm, out_hbm, idx_v, rows, gsem, wsem):
    wid = lax.axis_index("s") * NC + lax.axis_index("c")
    pltpu.sync_copy(tok_hbm.at[wid], idx_v)
    base = wid * TOK_PER_W

    gd = [None] * NCHUNK
    wd = [None] * NCHUNK

    def write_back(i):
        b = i % NBUF
        gd[i].wait()
        wd[i] = pltpu.async_copy(
            rows[b], out_hbm.at[pl.ds(base + i * CHUNK, CHUNK)], wsem[b]
        )

    for j in range(NCHUNK):
        b = j % NBUF
        if j >= NBUF:
            wd[j - NBUF].wait()
        gd[j] = pltpu.async_copy(p_hbm.at[idx_v.at[j]], rows[b], gsem[b])
        if j >= LAG:
            write_back(j - LAG)
    for i in range(NCHUNK - LAG, NCHUNK):
        write_back(i)
    for i in range(NCHUNK - NBUF, NCHUNK):
        wd[i].wait()


def kernel(tokens, table, W, b):
    P = _project_table(table, W, b)
    tok3 = tokens.astype(jnp.int32).reshape(NW, NCHUNK, CHUNK)
    return _sc_gather(P, tok3)


# speedup vs baseline: 5.3769x; 5.3769x over previous
import functools

import jax
import jax.numpy as jnp
from jax import lax
from jax.experimental import pallas as pl
from jax.experimental.pallas import tpu as pltpu
from jax.experimental.pallas import tpu_sc as plsc

K = 8
D = 128
T = 262144

NC = 2
NS = 16
NW = NC * NS
TOK_PER_W = T // NW
CHUNK = 128
NCHUNK = TOK_PER_W // CHUNK


def _proj_body(table_ref, w_ref, b_ref, out_ref):
    out_ref[...] = (
        jnp.dot(table_ref[...], w_ref[...], preferred_element_type=jnp.float32)
        + b_ref[...]
    )


def _project_table(table, W, b):
    return pl.pallas_call(
        _proj_body,
        out_shape=jax.ShapeDtypeStruct((K, D), jnp.float32),
    )(table, W, b.reshape(1, D))


_sc_mesh = plsc.VectorSubcoreMesh(
    core_axis_name="c", subcore_axis_name="s", num_cores=NC, num_subcores=NS
)

NBUF = 4
LAG = 2


@functools.partial(
    pl.kernel,
    out_type=jax.ShapeDtypeStruct((T, D), jnp.float32),
    mesh=_sc_mesh,
    scratch_types=[
        pltpu.VMEM((NCHUNK, CHUNK), jnp.int32),
        [pltpu.VMEM((CHUNK, D), jnp.float32)] * NBUF,
        [pltpu.SemaphoreType.DMA] * NBUF,
        [pltpu.SemaphoreType.DMA] * NBUF,
    ],
)
def _sc_gather(p_hbm, tok_hbm, out_hbm, idx_v, rows, gsem, wsem):
    wid = lax.axis_index("s") * NC + lax.axis_index("c")
    pltpu.sync_copy(tok_hbm.at[wid], idx_v)
    base = wid * TOK_PER_W

    off = (wid * K).astype(jnp.int32)

    def add_off(i, carry):
        r = i // (CHUNK // 16)
        c = (i % (CHUNK // 16)) * 16
        idx_v[r, pl.ds(c, 16)] = idx_v[r, pl.ds(c, 16)] + off
        return carry

    lax.fori_loop(0, NCHUNK * (CHUNK // 16), add_off, 0)

    gd = [None] * NCHUNK
    wd = [None] * NCHUNK

    def write_back(i):
        b = i % NBUF
        gd[i].wait()
        wd[i] = pltpu.async_copy(
            rows[b], out_hbm.at[pl.ds(base + i * CHUNK, CHUNK)], wsem[b]
        )

    for j in range(NCHUNK):
        b = j % NBUF
        if j >= NBUF:
            wd[j - NBUF].wait()
        gd[j] = pltpu.async_copy(p_hbm.at[idx_v.at[j]], rows[b], gsem[b])
        if j >= LAG:
            write_back(j - LAG)
    for i in range(NCHUNK - LAG, NCHUNK):
        write_back(i)
    for i in range(NCHUNK - NBUF, NCHUNK):
        wd[i].wait()


def kernel(tokens, table, W, b):
    P = _project_table(table, W, b)
    p_rep = jnp.tile(P, (NW, 1))
    tok3 = tokens.astype(jnp.int32).reshape(NW, NCHUNK, CHUNK)
    return _sc_gather(p_rep, tok3)


# speedup vs baseline: 19.9677x; 3.7136x over previous
import functools

import jax
import jax.numpy as jnp
from jax import lax
from jax.experimental import pallas as pl
from jax.experimental.pallas import tpu as pltpu
from jax.experimental.pallas import tpu_sc as plsc

K = 8
D = 128
T = 262144

NC = 2
NS = 16
NW = NC * NS
TOK_PER_W = T // NW
CHUNK = 128
NCHUNK = TOK_PER_W // CHUNK


def _proj_body(table_ref, w_ref, b_ref, out_ref):
    out_ref[...] = (
        jnp.dot(table_ref[...], w_ref[...], preferred_element_type=jnp.float32)
        + b_ref[...]
    )


def _project_table(table, W, b):
    return pl.pallas_call(
        _proj_body,
        out_shape=jax.ShapeDtypeStruct((K, D), jnp.float32),
    )(table, W, b.reshape(1, D))


_sc_mesh = plsc.VectorSubcoreMesh(
    core_axis_name="c", subcore_axis_name="s", num_cores=NC, num_subcores=NS
)

NBUF = 4
LAG = 2


@functools.partial(
    pl.kernel,
    out_type=jax.ShapeDtypeStruct((T, D), jnp.float32),
    mesh=_sc_mesh,
    scratch_types=[
        pltpu.VMEM((NCHUNK, CHUNK), jnp.int32),
        [pltpu.VMEM((CHUNK, D), jnp.float32)] * NBUF,
        pltpu.VMEM_SHARED((NS * K, D), jnp.float32),
        [pltpu.SemaphoreType.DMA] * NBUF,
        [pltpu.SemaphoreType.DMA] * NBUF,
    ],
)
def _sc_gather(p_hbm, tok_hbm, out_hbm, idx_v, rows, pshared, gsem, wsem):
    sid = lax.axis_index("s")
    wid = sid * NC + lax.axis_index("c")
    pltpu.sync_copy(p_hbm, pshared.at[pl.ds(sid * K, K)])
    pltpu.sync_copy(tok_hbm.at[wid], idx_v)
    plsc.subcore_barrier()
    base = wid * TOK_PER_W

    off = (sid * K).astype(jnp.int32)

    def add_off(i, carry):
        r = i // (CHUNK // 16)
        c = (i % (CHUNK // 16)) * 16
        idx_v[r, pl.ds(c, 16)] = idx_v[r, pl.ds(c, 16)] + off
        return carry

    lax.fori_loop(0, NCHUNK * (CHUNK // 16), add_off, 0)

    gd = [None] * NCHUNK
    wd = [None] * NCHUNK

    def write_back(i):
        b = i % NBUF
        gd[i].wait()
        wd[i] = pltpu.async_copy(
            rows[b], out_hbm.at[pl.ds(base + i * CHUNK, CHUNK)], wsem[b]
        )

    for j in range(NCHUNK):
        b = j % NBUF
        if j >= NBUF:
            wd[j - NBUF].wait()
        gd[j] = pltpu.async_copy(pshared.at[idx_v.at[j]], rows[b], gsem[b])
        if j >= LAG:
            write_back(j - LAG)
    for i in range(NCHUNK - LAG, NCHUNK):
        write_back(i)
    for i in range(NCHUNK - NBUF, NCHUNK):
        wd[i].wait()


def kernel(tokens, table, W, b):
    P = _project_table(table, W, b)
    tok3 = tokens.astype(jnp.int32).reshape(NW, NCHUNK, CHUNK)
    return _sc_gather(P, tok3)
